# baseline (speedup 1.0000x reference)
import functools

import jax
import jax.numpy as jnp
from jax import lax
from jax.experimental import pallas as pl

B, N, S, K, D = 8, 4096, 512, 32, 128
M = B * S * K
CIN = 144
TM = 512
EPS = 1e-5
_HI = lax.Precision.HIGHEST


def _mlp1_body(x_ref, w_ref, b_ref, y_ref, acc_ref):
    i = pl.program_id(0)
    y = jnp.dot(x_ref[...], w_ref[...], precision=_HI,
                preferred_element_type=jnp.float32) + b_ref[...]
    y_ref[...] = y
    ps = jnp.sum(y, axis=0, keepdims=True)
    pss = jnp.sum(y * y, axis=0, keepdims=True)
    blk = jnp.concatenate([ps, pss, jnp.zeros((6, y.shape[1]), jnp.float32)], axis=0)

    @pl.when(i == 0)
    def _():
        acc_ref[...] = blk

    @pl.when(i > 0)
    def _():
        acc_ref[...] = acc_ref[...] + blk


def _bn_scale_shift(acc_ref, g_ref, be_ref):
    m = acc_ref[0:1, :] / float(M)
    v = acc_ref[1:2, :] / float(M) - m * m
    sc = g_ref[...] / jnp.sqrt(v + EPS)
    sh = be_ref[...] - m * sc
    return sc, sh


def _mlp_mid_body(y_ref, acc_ref, g_ref, be_ref, w_ref, b_ref, y2_ref, acc2_ref):
    i = pl.program_id(0)
    sc, sh = _bn_scale_shift(acc_ref, g_ref, be_ref)
    h = jnp.maximum(y_ref[...] * sc + sh, 0.0)
    y2 = jnp.dot(h, w_ref[...], precision=_HI,
                 preferred_element_type=jnp.float32) + b_ref[...]
    y2_ref[...] = y2
    ps = jnp.sum(y2, axis=0, keepdims=True)
    pss = jnp.sum(y2 * y2, axis=0, keepdims=True)
    blk = jnp.concatenate([ps, pss, jnp.zeros((6, y2.shape[1]), jnp.float32)], axis=0)

    @pl.when(i == 0)
    def _():
        acc2_ref[...] = blk

    @pl.when(i > 0)
    def _():
        acc2_ref[...] = acc2_ref[...] + blk


def _bn3_max_body(y_ref, acc_ref, g_ref, be_ref, o_ref):
    sc, sh = _bn_scale_shift(acc_ref, g_ref, be_ref)
    h = jnp.maximum(y_ref[...] * sc + sh, 0.0)
    o_ref[...] = jnp.max(h.reshape(TM // K, K, h.shape[1]), axis=1)


def _row(v):
    return v.reshape(1, -1)


def _mlp_chain(x_flat, W1, b1, g1, be1, W2, b2, g2, be2, W3, b3, g3, be3):
    grid = (M // TM,)
    w1 = jnp.zeros((CIN, 128), jnp.float32).at[:131, :].set(W1.T)

    y1, acc1 = pl.pallas_call(
        _mlp1_body,
        grid=grid,
        in_specs=[
            pl.BlockSpec((TM, CIN), lambda i: (i, 0)),
            pl.BlockSpec((CIN, 128), lambda i: (0, 0)),
            pl.BlockSpec((1, 128), lambda i: (0, 0)),
        ],
        out_specs=[
            pl.BlockSpec((TM, 128), lambda i: (i, 0)),
            pl.BlockSpec((8, 128), lambda i: (0, 0)),
        ],
        out_shape=[
            jax.ShapeDtypeStruct((M, 128), jnp.float32),
            jax.ShapeDtypeStruct((8, 128), jnp.float32),
        ],
    )(x_flat, w1, _row(b1))

    def mid(y, acc, g, be, w, b, cout):
        return pl.pallas_call(
            _mlp_mid_body,
            grid=grid,
            in_specs=[
                pl.BlockSpec((TM, y.shape[1]), lambda i: (i, 0)),
                pl.BlockSpec((8, y.shape[1]), lambda i: (0, 0)),
                pl.BlockSpec((1, y.shape[1]), lambda i: (0, 0)),
                pl.BlockSpec((1, y.shape[1]), lambda i: (0, 0)),
                pl.BlockSpec((y.shape[1], cout), lambda i: (0, 0)),
                pl.BlockSpec((1, cout), lambda i: (0, 0)),
            ],
            out_specs=[
                pl.BlockSpec((TM, cout), lambda i: (i, 0)),
                pl.BlockSpec((8, cout), lambda i: (0, 0)),
            ],
            out_shape=[
                jax.ShapeDtypeStruct((M, cout), jnp.float32),
                jax.ShapeDtypeStruct((8, cout), jnp.float32),
            ],
        )(y, acc, _row(g), _row(be), w, _row(b))

    y2, acc2 = mid(y1, acc1, g1, be1, W2.T, b2, 128)
    y3, acc3 = mid(y2, acc2, g2, be2, W3.T, b3, 256)

    pooled = pl.pallas_call(
        _bn3_max_body,
        grid=grid,
        in_specs=[
            pl.BlockSpec((TM, 256), lambda i: (i, 0)),
            pl.BlockSpec((8, 256), lambda i: (0, 0)),
            pl.BlockSpec((1, 256), lambda i: (0, 0)),
            pl.BlockSpec((1, 256), lambda i: (0, 0)),
        ],
        out_specs=pl.BlockSpec((TM // K, 256), lambda i: (i, 0)),
        out_shape=jax.ShapeDtypeStruct((B * S, 256), jnp.float32),
    )(y3, acc3, _row(g3), _row(be3))
    return pooled


def kernel(xyz, points, sample_idx, W1, b1, g1, be1, W2, b2, g2, be2, W3, b3, g3, be3):
    xyz_t = jnp.transpose(xyz, (0, 2, 1))
    pts_t = jnp.transpose(points, (0, 2, 1))

    new_xyz = jnp.take_along_axis(xyz_t, sample_idx[..., None], axis=1)
    d = -2.0 * jnp.einsum('bnc,bmc->bnm', new_xyz, xyz_t)
    d = d + jnp.sum(new_xyz ** 2, axis=-1)[:, :, None]
    d = d + jnp.sum(xyz_t ** 2, axis=-1)[:, None, :]
    _, idx = lax.top_k(-d, K)

    grouped_xyz = jnp.take_along_axis(xyz_t[:, None, :, :], idx[..., None], axis=2)
    grouped_pts = jnp.take_along_axis(pts_t[:, None, :, :], idx[..., None], axis=2)
    gx_norm = grouped_xyz - new_xyz[:, :, None, :]

    x = jnp.concatenate([
        gx_norm.reshape(M, 3),
        grouped_pts.reshape(M, D),
        jnp.zeros((M, CIN - 3 - D), jnp.float32),
    ], axis=-1)

    pooled = _mlp_chain(x, W1, b1, g1, be1, W2, b2, g2, be2, W3, b3, g3, be3)
    new_points = pooled.reshape(B, S, 256).transpose(0, 2, 1)
    new_xyz_out = jnp.transpose(new_xyz, (0, 2, 1))
    return (new_xyz_out, new_points, grouped_xyz, sample_idx)


# speedup vs baseline: 1.9281x; 1.9281x over previous
import functools

import jax
import jax.numpy as jnp
from jax import lax
from jax.experimental import pallas as pl
from jax.experimental.pallas import tpu as pltpu
from jax.experimental.pallas import tpu_sc as plsc

B, N, S, K, D = 8, 4096, 512, 32, 128
M = B * S * K
TM = 512
G_PER_T = TM // K
EPS = 1e-5
_HI = lax.Precision.HIGHEST

NWORK = 32
RPW = M // NWORK
CH = 128
NCH = RPW // CH
WPB = NWORK // B


def _prep_body(p_ref, w_ref, q_ref):
    q_ref[0] = lax.dot_general(
        p_ref[0], w_ref[...], (((0,), (1,)), ((), ())),
        precision=_HI, preferred_element_type=jnp.float32)


def _prep_qt(points, W1b):
    return pl.pallas_call(
        _prep_body,
        grid=(B,),
        in_specs=[
            pl.BlockSpec((1, D, N), lambda b: (b, 0, 0)),
            pl.BlockSpec((128, D), lambda b: (0, 0)),
        ],
        out_specs=pl.BlockSpec((1, N, 128), lambda b: (b, 0, 0)),
        out_shape=jax.ShapeDtypeStruct((B, N, 128), jnp.float32),
    )(points, W1b)


CT = 256


def _sc_gather_body(tab, gidx, gout, idx_v, buf0, buf1, sem0, sem1):
    wid = lax.axis_index("s") * 2 + lax.axis_index("c")
    pltpu.sync_copy(gidx.at[wid], idx_v)
    bufs = (buf0, buf1)
    sems = (sem0, sem1)
    cp = pltpu.async_copy(tab.at[idx_v.at[0]], buf0, sem0)

    def chunk(jj, carry):
        for p in range(2):
            j = jj * 2 + p
            nxt = pltpu.async_copy(tab.at[idx_v.at[j + 1]], bufs[1 - p],
                                   sems[1 - p])
            pltpu.make_async_copy(tab.at[idx_v.at[j]], bufs[p], sems[p]).wait()
            pltpu.sync_copy(bufs[p], gout.at[pl.ds(wid * RPW + j * CH, CH)])
            del nxt
        return carry

    lax.fori_loop(0, (NCH - 2) // 2, chunk, 0)
    pltpu.async_copy(tab.at[idx_v.at[NCH - 1]], buf1, sem1)
    pltpu.make_async_copy(tab.at[idx_v.at[NCH - 2]], buf0, sem0).wait()
    pltpu.sync_copy(buf0, gout.at[pl.ds(wid * RPW + (NCH - 2) * CH, CH)])
    pltpu.make_async_copy(tab.at[idx_v.at[NCH - 1]], buf1, sem1).wait()
    pltpu.sync_copy(buf1, gout.at[pl.ds(wid * RPW + (NCH - 1) * CH, CH)])


def _sc_gather(table, gidx):
    mesh = plsc.VectorSubcoreMesh(core_axis_name="c", subcore_axis_name="s")
    return pl.kernel(
        _sc_gather_body,
        out_type=jax.ShapeDtypeStruct((M, CT), jnp.float32),
        mesh=mesh,
        scratch_types=[
            pltpu.VMEM((NCH, CH), jnp.int32),
            pltpu.VMEM((CH, CT), jnp.float32),
            pltpu.VMEM((CH, CT), jnp.float32),
            pltpu.SemaphoreType.DMA,
            pltpu.SemaphoreType.DMA,
        ],
    )(table, gidx)


def _mlp1_body(g_ref, nx_ref, wa_ref, b_ref, y_ref, acc_ref):
    i = pl.program_id(0)
    gx = g_ref[:, D:D + 3]
    y = g_ref[:, :D] + jnp.dot(gx, wa_ref[...], precision=_HI,
                               preferred_element_type=jnp.float32)
    cb = b_ref[...] - jnp.dot(nx_ref[...], wa_ref[...], precision=_HI,
                              preferred_element_type=jnp.float32)
    r = lax.broadcasted_iota(jnp.int32, (TM, G_PER_T), 0) // K
    c = lax.broadcasted_iota(jnp.int32, (TM, G_PER_T), 1)
    e = (r == c).astype(jnp.float32)
    y = y + jnp.dot(e, cb, precision=_HI, preferred_element_type=jnp.float32)
    y_ref[...] = y
    ps = jnp.sum(y, axis=0, keepdims=True)
    pss = jnp.sum(y * y, axis=0, keepdims=True)
    blk = jnp.concatenate([ps, pss, jnp.zeros((6, y.shape[1]), jnp.float32)], axis=0)

    @pl.when(i == 0)
    def _():
        acc_ref[...] = blk

    @pl.when(i > 0)
    def _():
        acc_ref[...] = acc_ref[...] + blk


def _bn_scale_shift(acc_ref, g_ref, be_ref):
    m = acc_ref[0:1, :] / float(M)
    v = acc_ref[1:2, :] / float(M) - m * m
    sc = g_ref[...] / jnp.sqrt(v + EPS)
    sh = be_ref[...] - m * sc
    return sc, sh


def _mlp_mid_body(y_ref, acc_ref, g_ref, be_ref, w_ref, b_ref, y2_ref, acc2_ref):
    i = pl.program_id(0)
    sc, sh = _bn_scale_shift(acc_ref, g_ref, be_ref)
    h = jnp.maximum(y_ref[...] * sc + sh, 0.0)
    y2 = jnp.dot(h, w_ref[...], precision=_HI,
                 preferred_element_type=jnp.float32) + b_ref[...]
    y2_ref[...] = y2
    ps = jnp.sum(y2, axis=0, keepdims=True)
    pss = jnp.sum(y2 * y2, axis=0, keepdims=True)
    blk = jnp.concatenate([ps, pss, jnp.zeros((6, y2.shape[1]), jnp.float32)], axis=0)

    @pl.when(i == 0)
    def _():
        acc2_ref[...] = blk

    @pl.when(i > 0)
    def _():
        acc2_ref[...] = acc2_ref[...] + blk


def _bn3_max_body(y_ref, acc_ref, g_ref, be_ref, o_ref):
    sc, sh = _bn_scale_shift(acc_ref, g_ref, be_ref)
    h = jnp.maximum(y_ref[...] * sc + sh, 0.0)
    o_ref[...] = jnp.max(h.reshape(G_PER_T, K, h.shape[1]), axis=1)


def _row(v):
    return v.reshape(1, -1)


def _mlp_chain(g_flat, nxyz_flat, W1, b1, g1, be1, W2, b2, g2, be2, W3, b3, g3, be3):
    grid = (M // TM,)
    wa = W1[:, :3].T

    y1, acc1 = pl.pallas_call(
        _mlp1_body,
        grid=grid,
        in_specs=[
            pl.BlockSpec((TM, CT), lambda i: (i, 0)),
            pl.BlockSpec((G_PER_T, 3), lambda i: (i, 0)),
            pl.BlockSpec((3, 128), lambda i: (0, 0)),
            pl.BlockSpec((1, 128), lambda i: (0, 0)),
        ],
        out_specs=[
            pl.BlockSpec((TM, 128), lambda i: (i, 0)),
            pl.BlockSpec((8, 128), lambda i: (0, 0)),
        ],
        out_shape=[
            jax.ShapeDtypeStruct((M, 128), jnp.float32),
            jax.ShapeDtypeStruct((8, 128), jnp.float32),
        ],
    )(g_flat, nxyz_flat, wa, _row(b1))

    def mid(y, acc, g, be, w, b, cout):
        return pl.pallas_call(
            _mlp_mid_body,
            grid=grid,
            in_specs=[
                pl.BlockSpec((TM, y.shape[1]), lambda i: (i, 0)),
                pl.BlockSpec((8, y.shape[1]), lambda i: (0, 0)),
                pl.BlockSpec((1, y.shape[1]), lambda i: (0, 0)),
                pl.BlockSpec((1, y.shape[1]), lambda i: (0, 0)),
                pl.BlockSpec((y.shape[1], cout), lambda i: (0, 0)),
                pl.BlockSpec((1, cout), lambda i: (0, 0)),
            ],
            out_specs=[
                pl.BlockSpec((TM, cout), lambda i: (i, 0)),
                pl.BlockSpec((8, cout), lambda i: (0, 0)),
            ],
            out_shape=[
                jax.ShapeDtypeStruct((M, cout), jnp.float32),
                jax.ShapeDtypeStruct((8, cout), jnp.float32),
            ],
        )(y, acc, _row(g), _row(be), w, _row(b))

    y2, acc2 = mid(y1, acc1, g1, be1, W2.T, b2, 128)
    y3, acc3 = mid(y2, acc2, g2, be2, W3.T, b3, 256)

    pooled = pl.pallas_call(
        _bn3_max_body,
        grid=grid,
        in_specs=[
            pl.BlockSpec((TM, 256), lambda i: (i, 0)),
            pl.BlockSpec((8, 256), lambda i: (0, 0)),
            pl.BlockSpec((1, 256), lambda i: (0, 0)),
            pl.BlockSpec((1, 256), lambda i: (0, 0)),
        ],
        out_specs=pl.BlockSpec((G_PER_T, 256), lambda i: (i, 0)),
        out_shape=jax.ShapeDtypeStruct((B * S, 256), jnp.float32),
    )(y3, acc3, _row(g3), _row(be3))
    return pooled


def kernel(xyz, points, sample_idx, W1, b1, g1, be1, W2, b2, g2, be2, W3, b3, g3, be3):
    xyz_t = jnp.transpose(xyz, (0, 2, 1))

    new_xyz = jnp.take_along_axis(xyz_t, sample_idx[..., None], axis=1)
    d = -2.0 * jnp.einsum('bnc,bmc->bnm', new_xyz, xyz_t)
    d = d + jnp.sum(new_xyz ** 2, axis=-1)[:, :, None]
    d = d + jnp.sum(xyz_t ** 2, axis=-1)[:, None, :]
    _, idx = lax.top_k(-d, K)

    qt = _prep_qt(points, W1[:, 3:])
    gidx = idx + (jnp.arange(B, dtype=idx.dtype) * N)[:, None, None]
    table = jnp.concatenate([
        qt, xyz_t, jnp.zeros((B, N, CT - D - 3), jnp.float32)], axis=-1)
    gfl = _sc_gather(table.reshape(B * N, CT),
                     gidx.reshape(NWORK, NCH, CH).astype(jnp.int32))

    grouped_xyz = gfl[:, D:D + 3].reshape(B, S, K, 3)

    pooled = _mlp_chain(gfl, new_xyz.reshape(B * S, 3),
                        W1, b1, g1, be1, W2, b2, g2, be2, W3, b3, g3, be3)
    new_points = pooled.reshape(B, S, 256).transpose(0, 2, 1)
    new_xyz_out = jnp.transpose(new_xyz, (0, 2, 1))
    return (new_xyz_out, new_points, grouped_xyz, sample_idx)


# speedup vs baseline: 4.0569x; 2.1041x over previous
import functools

import jax
import jax.numpy as jnp
from jax import lax
from jax.experimental import pallas as pl
from jax.experimental.pallas import tpu as pltpu
from jax.experimental.pallas import tpu_sc as plsc

B, N, S, K, D = 8, 4096, 512, 32, 128
M = B * S * K
TM = 512
G_PER_T = TM // K
EPS = 1e-5
_HI = lax.Precision.HIGHEST

NWORK = 32
RPW = M // NWORK
CH = 128
NCH = RPW // CH
WPB = NWORK // B


def _prep_body(p_ref, w_ref, q_ref):
    q_ref[0] = lax.dot_general(
        p_ref[0], w_ref[...], (((0,), (1,)), ((), ())),
        precision=_HI, preferred_element_type=jnp.float32)


def _prep_qt(points, W1b):
    return pl.pallas_call(
        _prep_body,
        grid=(B,),
        in_specs=[
            pl.BlockSpec((1, D, N), lambda b: (b, 0, 0)),
            pl.BlockSpec((128, D), lambda b: (0, 0)),
        ],
        out_specs=pl.BlockSpec((1, N, 128), lambda b: (b, 0, 0)),
        out_shape=jax.ShapeDtypeStruct((B, N, 128), jnp.float32),
    )(points, W1b)


CT = 256


def _sc_gather_body(tab, gidx, gout, idx_v, buf0, buf1, sem0, sem1):
    wid = lax.axis_index("s") * 2 + lax.axis_index("c")
    pltpu.sync_copy(gidx.at[wid], idx_v)
    bufs = (buf0, buf1)
    sems = (sem0, sem1)
    cp = pltpu.async_copy(tab.at[idx_v.at[0]], buf0, sem0)

    def chunk(jj, carry):
        for p in range(2):
            j = jj * 2 + p
            nxt = pltpu.async_copy(tab.at[idx_v.at[j + 1]], bufs[1 - p],
                                   sems[1 - p])
            pltpu.make_async_copy(tab.at[idx_v.at[j]], bufs[p], sems[p]).wait()
            pltpu.sync_copy(bufs[p], gout.at[pl.ds(wid * RPW + j * CH, CH)])
            del nxt
        return carry

    lax.fori_loop(0, (NCH - 2) // 2, chunk, 0)
    pltpu.async_copy(tab.at[idx_v.at[NCH - 1]], buf1, sem1)
    pltpu.make_async_copy(tab.at[idx_v.at[NCH - 2]], buf0, sem0).wait()
    pltpu.sync_copy(buf0, gout.at[pl.ds(wid * RPW + (NCH - 2) * CH, CH)])
    pltpu.make_async_copy(tab.at[idx_v.at[NCH - 1]], buf1, sem1).wait()
    pltpu.sync_copy(buf1, gout.at[pl.ds(wid * RPW + (NCH - 1) * CH, CH)])


def _sc_gather(table, gidx):
    mesh = plsc.VectorSubcoreMesh(core_axis_name="c", subcore_axis_name="s")
    return pl.kernel(
        _sc_gather_body,
        out_type=jax.ShapeDtypeStruct((M, CT), jnp.float32),
        mesh=mesh,
        scratch_types=[
            pltpu.VMEM((NCH, CH), jnp.int32),
            pltpu.VMEM((CH, CT), jnp.float32),
            pltpu.VMEM((CH, CT), jnp.float32),
            pltpu.SemaphoreType.DMA,
            pltpu.SemaphoreType.DMA,
        ],
    )(table, gidx)


ROWS_PW = (B * S) // NWORK
NV = N // 16
_BIG = 1 << 20


def _hist_zero(hist):
    z = jnp.zeros((16,), jnp.int32)

    def zb(zi, carry):
        for u in range(8):
            hist[pl.ds((zi * 8 + u) * 16, 16)] = z
        return carry

    lax.fori_loop(0, NV // 8, zb, 0)


def _cum_thresh(hist, base0):
    iota = lax.iota(jnp.int32, 16)

    def grp(g, carry):
        base, tmin = carry
        acc = hist[pl.ds(g * 16, 16)]
        for l in range(1, 16):
            acc = acc + hist[pl.ds(l * 256 + g * 16, 16)]
        cum = plsc.cumsum(acc) + base
        cand = jnp.where(cum >= K, g * 16 + iota, _BIG)
        return (base + jnp.sum(acc), jnp.minimum(tmin, jnp.min(cand)))

    _, t = lax.fori_loop(0, 16, grp, (base0, _BIG))
    return t


def _sc_topk_body(dflat, iout, dbuf0, dbuf1, mbuf, hist, cva, cia, cvb, cib,
                  fval, fidx, sem0, sem1):
    wid = lax.axis_index("s") * 2 + lax.axis_index("c")
    row0 = wid * ROWS_PW
    iota = lax.iota(jnp.int32, 16)
    ones = jnp.ones((16,), jnp.int32)
    lane256 = iota * 256

    pltpu.async_copy(dflat.at[pl.ds(row0 * N, N)], dbuf0, sem0)
    pltpu.async_copy(dflat.at[pl.ds((row0 + 1) * N, N)], dbuf1, sem1)

    def do_row(r, dbuf, sem):
        row = row0 + r
        pltpu.make_async_copy(dflat.at[pl.ds(row * N, N)], dbuf, sem).wait()

        _hist_zero(hist)

        def p1(vi, carry):
            for u in range(8):
                v = vi * 8 + u
                f = plsc.bitcast(dbuf[pl.ds(v * 16, 16)], jnp.int32)
                m = f ^ (lax.shift_right_arithmetic(f, 31)
                         & jnp.int32(0x7FFFFFFF))
                mbuf[pl.ds(v * 16, 16)] = m
                buck = lax.shift_right_arithmetic(m, 24) + 128
                plsc.addupdate_scatter(hist, [lane256 + buck], ones)
            return carry

        lax.fori_loop(0, NV // 8, p1, 0)

        @pl.when(r + 2 < ROWS_PW)
        def _():
            pltpu.async_copy(dflat.at[pl.ds((row + 2) * N, N)], dbuf, sem)

        t1 = _cum_thresh(hist, jnp.int32(0))

        ub = ((t1 - 127) << 24) - 1

        def coll(vi, off):
            for u in range(8):
                v = vi * 8 + u
                m = mbuf[pl.ds(v * 16, 16)]
                msk = m <= ub
                pref = plsc.cumsum(msk.astype(jnp.int32))
                pos = pref + (off - 1)
                plsc.store_scatter(cva, [pos], m, mask=msk)
                plsc.store_scatter(cia, [pos], v * 16 + iota, mask=msk)
                off = off + jnp.sum(msk.astype(jnp.int32))
            return off

        c = lax.fori_loop(0, NV // 8, coll, jnp.int32(0))

        pr = (t1 - 128) << 24
        bufs = ((cva, cia, cvb, cib), (cvb, cib, cva, cia),
                (cva, cia, cvb, cib))
        for li, sh in enumerate((16, 8, 0)):
            src_v, src_i, dst_v, dst_i = bufs[li]
            um = jnp.int32(-1 << (sh + 8))
            _hist_zero(hist)

            def hl(v, nb, src_v=src_v, sh=sh, um=um, pr=pr, c=c):
                m = src_v[pl.ds(v * 16, 16)]
                valid = (v * 16 + iota) < c
                onp = ((m & um) == pr) & valid
                byte = lax.shift_right_logical(m, sh) & 0xFF
                plsc.addupdate_scatter(hist, [lane256 + byte], ones, mask=onp)
                return nb + jnp.sum(((m < pr) & valid).astype(jnp.int32))

            nv = (c + 15) >> 4
            nb = lax.fori_loop(0, nv, hl, jnp.int32(0))
            t = _cum_thresh(hist, nb)

            def cl(v, off, src_v=src_v, src_i=src_i, dst_v=dst_v,
                   dst_i=dst_i, sh=sh, um=um, pr=pr, c=c, t=t):
                m = src_v[pl.ds(v * 16, 16)]
                ix = src_i[pl.ds(v * 16, 16)]
                valid = (v * 16 + iota) < c
                onp = (m & um) == pr
                byte = lax.shift_right_logical(m, sh) & 0xFF
                msk = valid & ((m < pr) | (onp & (byte <= t)))
                pref = plsc.cumsum(msk.astype(jnp.int32))
                pos = pref + (off - 1)
                plsc.store_scatter(dst_v, [pos], m, mask=msk)
                plsc.store_scatter(dst_i, [pos], ix, mask=msk)
                return off + jnp.sum(msk.astype(jnp.int32))

            c = lax.fori_loop(0, nv, cl, jnp.int32(0))
            pr = pr | (t << sh)

        fin_v, fin_i = cvb, cib
        nv = (c + 15) >> 4

        def cnt(v, na):
            m = fin_v[pl.ds(v * 16, 16)]
            valid = (v * 16 + iota) < c
            return na + jnp.sum(((m < pr) & valid).astype(jnp.int32))

        na = lax.fori_loop(0, nv, cnt, jnp.int32(0))

        def fl(v, carry):
            offa, offt = carry
            m = fin_v[pl.ds(v * 16, 16)]
            ix = fin_i[pl.ds(v * 16, 16)]
            valid = (v * 16 + iota) < c
            mska = (m < pr) & valid
            mskt = (m == pr) & valid
            pa = plsc.cumsum(mska.astype(jnp.int32)) + (offa - 1)
            pt = plsc.cumsum(mskt.astype(jnp.int32)) + (na + offt - 1)
            pos = jnp.where(mska, pa, pt)
            msk = mska | (mskt & (pos < K))
            plsc.store_scatter(fval, [pos], m, mask=msk)
            plsc.store_scatter(fidx, [pos], ix, mask=msk)
            return (offa + jnp.sum(mska.astype(jnp.int32)),
                    offt + jnp.sum(mskt.astype(jnp.int32)))

        lax.fori_loop(0, nv, fl, (jnp.int32(0), jnp.int32(0)))

        k0, i0 = plsc.sort_key_val(fval[pl.ds(0, 16)], fidx[pl.ds(0, 16)])
        k1, i1 = plsc.sort_key_val(fval[pl.ds(16, 16)], fidx[pl.ds(16, 16)])
        k1r = lax.rev(k1, (0,))
        i1r = lax.rev(i1, (0,))
        sel = k0 <= k1r
        lok = jnp.where(sel, k0, k1r)
        lov = jnp.where(sel, i0, i1r)
        hik = jnp.where(sel, k1r, k0)
        hiv = jnp.where(sel, i1r, i0)
        _, lv = plsc.sort_key_val(lok, lov)
        _, hv = plsc.sort_key_val(hik, hiv)
        fidx[pl.ds(0, 16)] = lv
        fidx[pl.ds(16, 16)] = hv
        pltpu.sync_copy(fidx, iout.at[pl.ds(row * K, K)])

    def pair(q, carry):
        do_row(q * 2, dbuf0, sem0)
        do_row(q * 2 + 1, dbuf1, sem1)
        return carry

    lax.fori_loop(0, ROWS_PW // 2, pair, 0)


def _sc_topk(dflat):
    mesh = plsc.VectorSubcoreMesh(core_axis_name="c", subcore_axis_name="s")
    return pl.kernel(
        _sc_topk_body,
        out_type=jax.ShapeDtypeStruct((B * S * K,), jnp.int32),
        mesh=mesh,
        compiler_params=pltpu.CompilerParams(needs_layout_passes=False),
        scratch_types=[
            pltpu.VMEM((N,), jnp.float32),
            pltpu.VMEM((N,), jnp.float32),
            pltpu.VMEM((N,), jnp.int32),
            pltpu.VMEM((N,), jnp.int32),
            pltpu.VMEM((N,), jnp.int32),
            pltpu.VMEM((N,), jnp.int32),
            pltpu.VMEM((N,), jnp.int32),
            pltpu.VMEM((N,), jnp.int32),
            pltpu.VMEM((K,), jnp.int32),
            pltpu.VMEM((K,), jnp.int32),
            pltpu.SemaphoreType.DMA,
            pltpu.SemaphoreType.DMA,
        ],
    )(dflat)


def _mlp1_body(g_ref, nx_ref, wa_ref, b_ref, y_ref, acc_ref):
    i = pl.program_id(0)
    gx = g_ref[:, D:D + 3]
    y = g_ref[:, :D] + jnp.dot(gx, wa_ref[...], precision=_HI,
                               preferred_element_type=jnp.float32)
    cb = b_ref[...] - jnp.dot(nx_ref[...], wa_ref[...], precision=_HI,
                              preferred_element_type=jnp.float32)
    r = lax.broadcasted_iota(jnp.int32, (TM, G_PER_T), 0) // K
    c = lax.broadcasted_iota(jnp.int32, (TM, G_PER_T), 1)
    e = (r == c).astype(jnp.float32)
    y = y + jnp.dot(e, cb, precision=_HI, preferred_element_type=jnp.float32)
    y_ref[...] = y
    ps = jnp.sum(y, axis=0, keepdims=True)
    pss = jnp.sum(y * y, axis=0, keepdims=True)
    blk = jnp.concatenate([ps, pss, jnp.zeros((6, y.shape[1]), jnp.float32)], axis=0)

    @pl.when(i == 0)
    def _():
        acc_ref[...] = blk

    @pl.when(i > 0)
    def _():
        acc_ref[...] = acc_ref[...] + blk


def _bn_scale_shift(acc_ref, g_ref, be_ref):
    m = acc_ref[0:1, :] / float(M)
    v = acc_ref[1:2, :] / float(M) - m * m
    sc = g_ref[...] / jnp.sqrt(v + EPS)
    sh = be_ref[...] - m * sc
    return sc, sh


def _mlp_mid_body(y_ref, acc_ref, g_ref, be_ref, w_ref, b_ref, y2_ref, acc2_ref):
    i = pl.program_id(0)
    sc, sh = _bn_scale_shift(acc_ref, g_ref, be_ref)
    h = jnp.maximum(y_ref[...] * sc + sh, 0.0)
    y2 = jnp.dot(h, w_ref[...], precision=_HI,
                 preferred_element_type=jnp.float32) + b_ref[...]
    y2_ref[...] = y2
    ps = jnp.sum(y2, axis=0, keepdims=True)
    pss = jnp.sum(y2 * y2, axis=0, keepdims=True)
    blk = jnp.concatenate([ps, pss, jnp.zeros((6, y2.shape[1]), jnp.float32)], axis=0)

    @pl.when(i == 0)
    def _():
        acc2_ref[...] = blk

    @pl.when(i > 0)
    def _():
        acc2_ref[...] = acc2_ref[...] + blk


def _bn3_max_body(y_ref, acc_ref, g_ref, be_ref, o_ref):
    sc, sh = _bn_scale_shift(acc_ref, g_ref, be_ref)
    h = jnp.maximum(y_ref[...] * sc + sh, 0.0)
    o_ref[...] = jnp.max(h.reshape(G_PER_T, K, h.shape[1]), axis=1)


def _row(v):
    return v.reshape(1, -1)


def _mlp_chain(g_flat, nxyz_flat, W1, b1, g1, be1, W2, b2, g2, be2, W3, b3, g3, be3):
    grid = (M // TM,)
    wa = W1[:, :3].T

    y1, acc1 = pl.pallas_call(
        _mlp1_body,
        grid=grid,
        in_specs=[
            pl.BlockSpec((TM, CT), lambda i: (i, 0)),
            pl.BlockSpec((G_PER_T, 3), lambda i: (i, 0)),
            pl.BlockSpec((3, 128), lambda i: (0, 0)),
            pl.BlockSpec((1, 128), lambda i: (0, 0)),
        ],
        out_specs=[
            pl.BlockSpec((TM, 128), lambda i: (i, 0)),
            pl.BlockSpec((8, 128), lambda i: (0, 0)),
        ],
        out_shape=[
            jax.ShapeDtypeStruct((M, 128), jnp.float32),
            jax.ShapeDtypeStruct((8, 128), jnp.float32),
        ],
    )(g_flat, nxyz_flat, wa, _row(b1))

    def mid(y, acc, g, be, w, b, cout):
        return pl.pallas_call(
            _mlp_mid_body,
            grid=grid,
            in_specs=[
                pl.BlockSpec((TM, y.shape[1]), lambda i: (i, 0)),
                pl.BlockSpec((8, y.shape[1]), lambda i: (0, 0)),
                pl.BlockSpec((1, y.shape[1]), lambda i: (0, 0)),
                pl.BlockSpec((1, y.shape[1]), lambda i: (0, 0)),
                pl.BlockSpec((y.shape[1], cout), lambda i: (0, 0)),
                pl.BlockSpec((1, cout), lambda i: (0, 0)),
            ],
            out_specs=[
                pl.BlockSpec((TM, cout), lambda i: (i, 0)),
                pl.BlockSpec((8, cout), lambda i: (0, 0)),
            ],
            out_shape=[
                jax.ShapeDtypeStruct((M, cout), jnp.float32),
                jax.ShapeDtypeStruct((8, cout), jnp.float32),
            ],
        )(y, acc, _row(g), _row(be), w, _row(b))

    y2, acc2 = mid(y1, acc1, g1, be1, W2.T, b2, 128)
    y3, acc3 = mid(y2, acc2, g2, be2, W3.T, b3, 256)

    pooled = pl.pallas_call(
        _bn3_max_body,
        grid=grid,
        in_specs=[
            pl.BlockSpec((TM, 256), lambda i: (i, 0)),
            pl.BlockSpec((8, 256), lambda i: (0, 0)),
            pl.BlockSpec((1, 256), lambda i: (0, 0)),
            pl.BlockSpec((1, 256), lambda i: (0, 0)),
        ],
        out_specs=pl.BlockSpec((G_PER_T, 256), lambda i: (i, 0)),
        out_shape=jax.ShapeDtypeStruct((B * S, 256), jnp.float32),
    )(y3, acc3, _row(g3), _row(be3))
    return pooled


def kernel(xyz, points, sample_idx, W1, b1, g1, be1, W2, b2, g2, be2, W3, b3, g3, be3):
    xyz_t = jnp.transpose(xyz, (0, 2, 1))

    new_xyz = jnp.take_along_axis(xyz_t, sample_idx[..., None], axis=1)
    d = -2.0 * jnp.einsum('bnc,bmc->bnm', new_xyz, xyz_t)
    d = d + jnp.sum(new_xyz ** 2, axis=-1)[:, :, None]
    d = d + jnp.sum(xyz_t ** 2, axis=-1)[:, None, :]
    idx = _sc_topk(d.reshape(B * S * N)).reshape(B, S, K)

    qt = _prep_qt(points, W1[:, 3:])
    gidx = idx + (jnp.arange(B, dtype=idx.dtype) * N)[:, None, None]
    table = jnp.concatenate([
        qt, xyz_t, jnp.zeros((B, N, CT - D - 3), jnp.float32)], axis=-1)
    gfl = _sc_gather(table.reshape(B * N, CT),
                     gidx.reshape(NWORK, NCH, CH).astype(jnp.int32))

    grouped_xyz = gfl[:, D:D + 3].reshape(B, S, K, 3)

    pooled = _mlp_chain(gfl, new_xyz.reshape(B * S, 3),
                        W1, b1, g1, be1, W2, b2, g2, be2, W3, b3, g3, be3)
    new_points = pooled.reshape(B, S, 256).transpose(0, 2, 1)
    new_xyz_out = jnp.transpose(new_xyz, (0, 2, 1))
    return (new_xyz_out, new_points, grouped_xyz, sample_idx)


# speedup vs baseline: 4.1136x; 1.0140x over previous
import functools

import jax
import jax.numpy as jnp
from jax import lax
from jax.experimental import pallas as pl
from jax.experimental.pallas import tpu as pltpu
from jax.experimental.pallas import tpu_sc as plsc

B, N, S, K, D = 8, 4096, 512, 32, 128
M = B * S * K
TM = 512
G_PER_T = TM // K
EPS = 1e-5
_HI = lax.Precision.HIGHEST

NWORK = 32
RPW = M // NWORK
CH = 128
NCH = RPW // CH
WPB = NWORK // B


def _prep_body(p_ref, w_ref, q_ref):
    q_ref[0] = lax.dot_general(
        p_ref[0], w_ref[...], (((0,), (1,)), ((), ())),
        precision=_HI, preferred_element_type=jnp.float32)


def _prep_qt(points, W1b):
    return pl.pallas_call(
        _prep_body,
        grid=(B,),
        in_specs=[
            pl.BlockSpec((1, D, N), lambda b: (b, 0, 0)),
            pl.BlockSpec((128, D), lambda b: (0, 0)),
        ],
        out_specs=pl.BlockSpec((1, N, 128), lambda b: (b, 0, 0)),
        out_shape=jax.ShapeDtypeStruct((B, N, 128), jnp.float32),
    )(points, W1b)


CT = 256


def _sc_gather_body(tab, gidx, gout, idx_v, buf0, buf1, sem0, sem1):
    wid = lax.axis_index("s") * 2 + lax.axis_index("c")
    pltpu.sync_copy(gidx.at[wid], idx_v)
    bufs = (buf0, buf1)
    sems = (sem0, sem1)
    cp = pltpu.async_copy(tab.at[idx_v.at[0]], buf0, sem0)

    def chunk(jj, carry):
        for p in range(2):
            j = jj * 2 + p
            nxt = pltpu.async_copy(tab.at[idx_v.at[j + 1]], bufs[1 - p],
                                   sems[1 - p])
            pltpu.make_async_copy(tab.at[idx_v.at[j]], bufs[p], sems[p]).wait()
            pltpu.sync_copy(bufs[p], gout.at[pl.ds(wid * RPW + j * CH, CH)])
            del nxt
        return carry

    lax.fori_loop(0, (NCH - 2) // 2, chunk, 0)
    pltpu.async_copy(tab.at[idx_v.at[NCH - 1]], buf1, sem1)
    pltpu.make_async_copy(tab.at[idx_v.at[NCH - 2]], buf0, sem0).wait()
    pltpu.sync_copy(buf0, gout.at[pl.ds(wid * RPW + (NCH - 2) * CH, CH)])
    pltpu.make_async_copy(tab.at[idx_v.at[NCH - 1]], buf1, sem1).wait()
    pltpu.sync_copy(buf1, gout.at[pl.ds(wid * RPW + (NCH - 1) * CH, CH)])


def _sc_gather(table, gidx):
    mesh = plsc.VectorSubcoreMesh(core_axis_name="c", subcore_axis_name="s")
    return pl.kernel(
        _sc_gather_body,
        out_type=jax.ShapeDtypeStruct((M, CT), jnp.float32),
        mesh=mesh,
        scratch_types=[
            pltpu.VMEM((NCH, CH), jnp.int32),
            pltpu.VMEM((CH, CT), jnp.float32),
            pltpu.VMEM((CH, CT), jnp.float32),
            pltpu.SemaphoreType.DMA,
            pltpu.SemaphoreType.DMA,
        ],
    )(table, gidx)


ROWS_PW = (B * S) // NWORK
NV = N // 16
_BIG = 1 << 20


def _hist_zero(hist):
    z = jnp.zeros((16,), jnp.int32)

    def zb(zi, carry):
        for u in range(8):
            hist[pl.ds((zi * 8 + u) * 16, 16)] = z
        return carry

    lax.fori_loop(0, NV // 8, zb, 0)


def _popc(msk):
    return plsc.all_reduce_population_count(msk)


def _cum_thresh(hist):
    sg = []
    for g in range(16):
        acc = hist[pl.ds(g * 16, 16)]
        for l in range(1, 16):
            acc = acc + hist[pl.ds(l * 256 + g * 16, 16)]
        sg.append(acc)
    cums = [plsc.cumsum(a) for a in sg]
    tots = [jnp.sum(a) for a in sg]
    tmin = jnp.full((16,), _BIG, jnp.int32)
    base = jnp.int32(0)
    for g in range(16):
        m = (cums[g] + base) >= K
        f = plsc.all_reduce_ffs(m)
        cnt = _popc(m)
        cand = jnp.where(cnt > 0, g * 16 + f, _BIG)
        tmin = jnp.minimum(tmin, cand)
        base = base + tots[g]
    return tmin


def _zero16(h):
    z = jnp.zeros((16,), jnp.int32)
    for l in range(16):
        h[pl.ds(l * 16, 16)] = z


def _cum_thresh16(h, nb):
    acc = h[pl.ds(0, 16)]
    for l in range(1, 16):
        acc = acc + h[pl.ds(l * 16, 16)]
    cum = plsc.cumsum(acc) + nb
    return plsc.all_reduce_ffs(cum >= K)


def _sc_topk_body(dflat, iout, dbuf0, dbuf1, mbuf, hist, cva, cia, cvb, cib,
                  fval, fidx, sem0, sem1):
    wid = lax.axis_index("s") * 2 + lax.axis_index("c")
    row0 = wid * ROWS_PW
    iota = lax.iota(jnp.int32, 16)
    ones = jnp.ones((16,), jnp.int32)
    lane256 = iota * 256

    pltpu.async_copy(dflat.at[pl.ds(row0 * N, N)], dbuf0, sem0)
    pltpu.async_copy(dflat.at[pl.ds((row0 + 1) * N, N)], dbuf1, sem1)

    def do_row(r, dbuf, sem):
        row = row0 + r
        pltpu.make_async_copy(dflat.at[pl.ds(row * N, N)], dbuf, sem).wait()

        _hist_zero(hist)

        def p1(vi, carry):
            for u in range(8):
                v = vi * 8 + u
                f = plsc.bitcast(dbuf[pl.ds(v * 16, 16)], jnp.int32)
                m = f ^ (lax.shift_right_arithmetic(f, 31)
                         & jnp.int32(0x7FFFFFFF))
                mbuf[pl.ds(v * 16, 16)] = m
                buck = lax.shift_right_arithmetic(m, 24) + 128
                plsc.addupdate_scatter(hist, [lane256 + buck], ones)
            return carry

        lax.fori_loop(0, NV // 8, p1, 0)

        @pl.when(r + 2 < ROWS_PW)
        def _():
            pltpu.async_copy(dflat.at[pl.ds((row + 2) * N, N)], dbuf, sem)

        t1 = _cum_thresh(hist)

        ub = ((t1 - 127) << 24) - 1

        def coll(vi, off):
            for u in range(8):
                v = vi * 8 + u
                m = mbuf[pl.ds(v * 16, 16)]
                msk = m <= ub
                pref = plsc.cumsum(msk.astype(jnp.int32))
                pos = pref + (off - 1)
                plsc.store_scatter(cva, [pos], m, mask=msk)
                plsc.store_scatter(cia, [pos], v * 16 + iota, mask=msk)
                off = off + _popc(msk)
            return off

        c = lax.fori_loop(0, NV // 8, coll, jnp.zeros((16,), jnp.int32))

        pr = (t1 - 128) << 24
        lane16 = iota * 16
        for li, sh in enumerate((20, 16, 12, 8, 4, 0)):
            if li % 2 == 0:
                src_v, src_i, dst_v, dst_i = cva, cia, cvb, cib
            else:
                src_v, src_i, dst_v, dst_i = cvb, cib, cva, cia
            um = jnp.int32(-1 << (sh + 4))
            _zero16(hist)
            nv = jnp.max(c) + 15 >> 4

            def hl(v, nb, src_v=src_v, sh=sh, um=um, pr=pr, c=c):
                m = src_v[pl.ds(v * 16, 16)]
                valid = (v * 16 + iota) < c
                onp = ((m & um) == pr) & valid
                nib = lax.shift_right_logical(m, sh) & 0xF
                plsc.addupdate_scatter(hist, [lane16 + nib], ones, mask=onp)
                return nb + _popc((m < pr) & valid)

            nb = lax.fori_loop(0, nv, hl, jnp.zeros((16,), jnp.int32))
            t = _cum_thresh16(hist, nb)

            def cl(v, off, src_v=src_v, src_i=src_i, dst_v=dst_v,
                   dst_i=dst_i, sh=sh, um=um, pr=pr, c=c, t=t):
                m = src_v[pl.ds(v * 16, 16)]
                ix = src_i[pl.ds(v * 16, 16)]
                valid = (v * 16 + iota) < c
                onp = (m & um) == pr
                nib = lax.shift_right_logical(m, sh) & 0xF
                msk = valid & ((m < pr) | (onp & (nib <= t)))
                pref = plsc.cumsum(msk.astype(jnp.int32))
                pos = pref + (off - 1)
                plsc.store_scatter(dst_v, [pos], m, mask=msk)
                plsc.store_scatter(dst_i, [pos], ix, mask=msk)
                return off + _popc(msk)

            c = lax.fori_loop(0, nv, cl, jnp.zeros((16,), jnp.int32))
            pr = pr | (t << sh)

        fin_v, fin_i = cva, cia
        nv = jnp.max(c) + 15 >> 4

        def cnt(v, na):
            m = fin_v[pl.ds(v * 16, 16)]
            valid = (v * 16 + iota) < c
            return na + _popc((m < pr) & valid)

        na = lax.fori_loop(0, nv, cnt, jnp.zeros((16,), jnp.int32))

        def fl(v, carry):
            offa, offt = carry
            m = fin_v[pl.ds(v * 16, 16)]
            ix = fin_i[pl.ds(v * 16, 16)]
            valid = (v * 16 + iota) < c
            mska = (m < pr) & valid
            mskt = (m == pr) & valid
            pa = plsc.cumsum(mska.astype(jnp.int32)) + (offa - 1)
            pt = plsc.cumsum(mskt.astype(jnp.int32)) + (na + offt - 1)
            pos = jnp.where(mska, pa, pt)
            msk = mska | (mskt & (pos < K))
            plsc.store_scatter(fval, [pos], m, mask=msk)
            plsc.store_scatter(fidx, [pos], ix, mask=msk)
            return (offa + _popc(mska), offt + _popc(mskt))

        lax.fori_loop(0, nv, fl, (jnp.zeros((16,), jnp.int32),
                                  jnp.zeros((16,), jnp.int32)))

        k0, i0 = plsc.sort_key_val(fval[pl.ds(0, 16)], fidx[pl.ds(0, 16)])
        k1, i1 = plsc.sort_key_val(fval[pl.ds(16, 16)], fidx[pl.ds(16, 16)])
        k1r = lax.rev(k1, (0,))
        i1r = lax.rev(i1, (0,))
        sel = k0 <= k1r
        lok = jnp.where(sel, k0, k1r)
        lov = jnp.where(sel, i0, i1r)
        hik = jnp.where(sel, k1r, k0)
        hiv = jnp.where(sel, i1r, i0)
        _, lv = plsc.sort_key_val(lok, lov)
        _, hv = plsc.sort_key_val(hik, hiv)
        fidx[pl.ds(0, 16)] = lv
        fidx[pl.ds(16, 16)] = hv
        pltpu.sync_copy(fidx, iout.at[pl.ds(row * K, K)])

    def pair(q, carry):
        do_row(q * 2, dbuf0, sem0)
        do_row(q * 2 + 1, dbuf1, sem1)
        return carry

    lax.fori_loop(0, ROWS_PW // 2, pair, 0)


def _sc_topk(dflat):
    mesh = plsc.VectorSubcoreMesh(core_axis_name="c", subcore_axis_name="s")
    return pl.kernel(
        _sc_topk_body,
        out_type=jax.ShapeDtypeStruct((B * S * K,), jnp.int32),
        mesh=mesh,
        compiler_params=pltpu.CompilerParams(needs_layout_passes=False),
        scratch_types=[
            pltpu.VMEM((N,), jnp.float32),
            pltpu.VMEM((N,), jnp.float32),
            pltpu.VMEM((N,), jnp.int32),
            pltpu.VMEM((N,), jnp.int32),
            pltpu.VMEM((N,), jnp.int32),
            pltpu.VMEM((N,), jnp.int32),
            pltpu.VMEM((N,), jnp.int32),
            pltpu.VMEM((N,), jnp.int32),
            pltpu.VMEM((K,), jnp.int32),
            pltpu.VMEM((K,), jnp.int32),
            pltpu.SemaphoreType.DMA,
            pltpu.SemaphoreType.DMA,
        ],
    )(dflat)


def _mlp1_body(g_ref, nx_ref, wa_ref, b_ref, y_ref, acc_ref):
    i = pl.program_id(0)
    gx = g_ref[:, D:D + 3]
    y = g_ref[:, :D] + jnp.dot(gx, wa_ref[...], precision=_HI,
                               preferred_element_type=jnp.float32)
    cb = b_ref[...] - jnp.dot(nx_ref[...], wa_ref[...], precision=_HI,
                              preferred_element_type=jnp.float32)
    r = lax.broadcasted_iota(jnp.int32, (TM, G_PER_T), 0) // K
    c = lax.broadcasted_iota(jnp.int32, (TM, G_PER_T), 1)
    e = (r == c).astype(jnp.float32)
    y = y + jnp.dot(e, cb, precision=_HI, preferred_element_type=jnp.float32)
    y_ref[...] = y
    ps = jnp.sum(y, axis=0, keepdims=True)
    pss = jnp.sum(y * y, axis=0, keepdims=True)
    blk = jnp.concatenate([ps, pss, jnp.zeros((6, y.shape[1]), jnp.float32)], axis=0)

    @pl.when(i == 0)
    def _():
        acc_ref[...] = blk

    @pl.when(i > 0)
    def _():
        acc_ref[...] = acc_ref[...] + blk


def _bn_scale_shift(acc_ref, g_ref, be_ref):
    m = acc_ref[0:1, :] / float(M)
    v = acc_ref[1:2, :] / float(M) - m * m
    sc = g_ref[...] / jnp.sqrt(v + EPS)
    sh = be_ref[...] - m * sc
    return sc, sh


def _mlp_mid_body(y_ref, acc_ref, g_ref, be_ref, w_ref, b_ref, y2_ref, acc2_ref):
    i = pl.program_id(0)
    sc, sh = _bn_scale_shift(acc_ref, g_ref, be_ref)
    h = jnp.maximum(y_ref[...] * sc + sh, 0.0)
    y2 = jnp.dot(h, w_ref[...], precision=_HI,
                 preferred_element_type=jnp.float32) + b_ref[...]
    y2_ref[...] = y2
    ps = jnp.sum(y2, axis=0, keepdims=True)
    pss = jnp.sum(y2 * y2, axis=0, keepdims=True)
    blk = jnp.concatenate([ps, pss, jnp.zeros((6, y2.shape[1]), jnp.float32)], axis=0)

    @pl.when(i == 0)
    def _():
        acc2_ref[...] = blk

    @pl.when(i > 0)
    def _():
        acc2_ref[...] = acc2_ref[...] + blk


def _bn3_max_body(y_ref, acc_ref, g_ref, be_ref, o_ref):
    sc, sh = _bn_scale_shift(acc_ref, g_ref, be_ref)
    h = jnp.maximum(y_ref[...] * sc + sh, 0.0)
    o_ref[...] = jnp.max(h.reshape(G_PER_T, K, h.shape[1]), axis=1)


def _row(v):
    return v.reshape(1, -1)


def _mlp_chain(g_flat, nxyz_flat, W1, b1, g1, be1, W2, b2, g2, be2, W3, b3, g3, be3):
    grid = (M // TM,)
    wa = W1[:, :3].T

    y1, acc1 = pl.pallas_call(
        _mlp1_body,
        grid=grid,
        in_specs=[
            pl.BlockSpec((TM, CT), lambda i: (i, 0)),
            pl.BlockSpec((G_PER_T, 3), lambda i: (i, 0)),
            pl.BlockSpec((3, 128), lambda i: (0, 0)),
            pl.BlockSpec((1, 128), lambda i: (0, 0)),
        ],
        out_specs=[
            pl.BlockSpec((TM, 128), lambda i: (i, 0)),
            pl.BlockSpec((8, 128), lambda i: (0, 0)),
        ],
        out_shape=[
            jax.ShapeDtypeStruct((M, 128), jnp.float32),
            jax.ShapeDtypeStruct((8, 128), jnp.float32),
        ],
    )(g_flat, nxyz_flat, wa, _row(b1))

    def mid(y, acc, g, be, w, b, cout):
        return pl.pallas_call(
            _mlp_mid_body,
            grid=grid,
            in_specs=[
                pl.BlockSpec((TM, y.shape[1]), lambda i: (i, 0)),
                pl.BlockSpec((8, y.shape[1]), lambda i: (0, 0)),
                pl.BlockSpec((1, y.shape[1]), lambda i: (0, 0)),
                pl.BlockSpec((1, y.shape[1]), lambda i: (0, 0)),
                pl.BlockSpec((y.shape[1], cout), lambda i: (0, 0)),
                pl.BlockSpec((1, cout), lambda i: (0, 0)),
            ],
            out_specs=[
                pl.BlockSpec((TM, cout), lambda i: (i, 0)),
                pl.BlockSpec((8, cout), lambda i: (0, 0)),
            ],
            out_shape=[
                jax.ShapeDtypeStruct((M, cout), jnp.float32),
                jax.ShapeDtypeStruct((8, cout), jnp.float32),
            ],
        )(y, acc, _row(g), _row(be), w, _row(b))

    y2, acc2 = mid(y1, acc1, g1, be1, W2.T, b2, 128)
    y3, acc3 = mid(y2, acc2, g2, be2, W3.T, b3, 256)

    pooled = pl.pallas_call(
        _bn3_max_body,
        grid=grid,
        in_specs=[
            pl.BlockSpec((TM, 256), lambda i: (i, 0)),
            pl.BlockSpec((8, 256), lambda i: (0, 0)),
            pl.BlockSpec((1, 256), lambda i: (0, 0)),
            pl.BlockSpec((1, 256), lambda i: (0, 0)),
        ],
        out_specs=pl.BlockSpec((G_PER_T, 256), lambda i: (i, 0)),
        out_shape=jax.ShapeDtypeStruct((B * S, 256), jnp.float32),
    )(y3, acc3, _row(g3), _row(be3))
    return pooled


def kernel(xyz, points, sample_idx, W1, b1, g1, be1, W2, b2, g2, be2, W3, b3, g3, be3):
    xyz_t = jnp.transpose(xyz, (0, 2, 1))

    new_xyz = jnp.take_along_axis(xyz_t, sample_idx[..., None], axis=1)
    d = -2.0 * jnp.einsum('bnc,bmc->bnm', new_xyz, xyz_t)
    d = d + jnp.sum(new_xyz ** 2, axis=-1)[:, :, None]
    d = d + jnp.sum(xyz_t ** 2, axis=-1)[:, None, :]
    idx = _sc_topk(d.reshape(B * S * N)).reshape(B, S, K)

    qt = _prep_qt(points, W1[:, 3:])
    gidx = idx + (jnp.arange(B, dtype=idx.dtype) * N)[:, None, None]
    table = jnp.concatenate([
        qt, xyz_t, jnp.zeros((B, N, CT - D - 3), jnp.float32)], axis=-1)
    gfl = _sc_gather(table.reshape(B * N, CT),
                     gidx.reshape(NWORK, NCH, CH).astype(jnp.int32))

    grouped_xyz = gfl[:, D:D + 3].reshape(B, S, K, 3)

    pooled = _mlp_chain(gfl, new_xyz.reshape(B * S, 3),
                        W1, b1, g1, be1, W2, b2, g2, be2, W3, b3, g3, be3)
    new_points = pooled.reshape(B, S, 256).transpose(0, 2, 1)
    new_xyz_out = jnp.transpose(new_xyz, (0, 2, 1))
    return (new_xyz_out, new_points, grouped_xyz, sample_idx)
